# Initial kernel scaffold; baseline (speedup 1.0000x reference)
import jax
import jax.numpy as jnp
from jax.experimental import pallas as pl


def kernel(input, embedding):
    raise NotImplementedError("write your pallas kernel here")


# speedup vs baseline: 2.9775x; 2.9775x over previous
import functools

import jax
import jax.numpy as jnp
from jax import lax
from jax.experimental import pallas as pl
from jax.experimental.pallas import tpu as pltpu
from jax.experimental.pallas import tpu_sc as plsc

N_EMBED = 128
NC = 2
NS = 16
NW = NC * NS
G = 128
NG = 50

_mesh = plsc.VectorSubcoreMesh(core_axis_name="c", subcore_axis_name="s")


@functools.partial(
    pl.kernel,
    out_type=jax.ShapeDtypeStruct((NW * NG * G, N_EMBED), jnp.float32),
    mesh=_mesh,
    scratch_types=[
        pltpu.VMEM((NG, G), jnp.int32),
        pltpu.VMEM((G, N_EMBED), jnp.float32),
        pltpu.SemaphoreType.DMA,
    ],
)
def _gather_kernel(idx_hbm, table_hbm, out_hbm, idx_v, rows_v, sem):
    wid = lax.axis_index("s") * NC + lax.axis_index("c")
    pltpu.sync_copy(idx_hbm.at[wid], idx_v)
    base = wid * (NG * G)

    def grp(j, carry):
        pltpu.async_copy(table_hbm.at[idx_v.at[j]], rows_v, sem).wait()
        pltpu.sync_copy(rows_v, out_hbm.at[pl.ds(base + j * G, G)])
        return carry

    lax.fori_loop(0, NG, grp, 0)


def kernel(input, embedding):
    B, H = input.shape
    idx = input.astype(jnp.int32).reshape(NW, NG, G)
    out = _gather_kernel(idx, embedding)
    return out.reshape(B, H, N_EMBED)


# speedup vs baseline: 3.3408x; 1.1220x over previous
import functools

import jax
import jax.numpy as jnp
from jax import lax
from jax.experimental import pallas as pl
from jax.experimental.pallas import tpu as pltpu
from jax.experimental.pallas import tpu_sc as plsc

N_EMBED = 128
NC = 2
NS = 16
NW = NC * NS
G = 128
NG = 50

_mesh = plsc.VectorSubcoreMesh(core_axis_name="c", subcore_axis_name="s")


NBUF = 5


@functools.partial(
    pl.kernel,
    out_type=jax.ShapeDtypeStruct((NW * NG * G, N_EMBED), jnp.float32),
    mesh=_mesh,
    scratch_types=[
        pltpu.VMEM((NG, G), jnp.int32),
        [pltpu.VMEM((G, N_EMBED), jnp.float32) for _ in range(NBUF)],
        [pltpu.SemaphoreType.DMA for _ in range(NBUF)],
        [pltpu.SemaphoreType.DMA for _ in range(NBUF)],
    ],
)
def _gather_kernel(idx_hbm, table_hbm, out_hbm, idx_v, bufs, gsems, psems):
    wid = lax.axis_index("s") * NC + lax.axis_index("c")
    pltpu.sync_copy(idx_hbm.at[wid], idx_v)
    base = wid * (NG * G)

    for b in range(NBUF):
        pltpu.async_copy(table_hbm.at[idx_v.at[b]], bufs[b], gsems[b])

    def outer(t, carry):
        for b in range(NBUF):
            j = t * NBUF + b
            pltpu.make_async_copy(table_hbm.at[idx_v.at[j]], bufs[b],
                                  gsems[b]).wait()
            put = pltpu.async_copy(bufs[b], out_hbm.at[pl.ds(base + j * G, G)],
                                   psems[b])
            put.wait()

            @pl.when(j + NBUF < NG)
            def _():
                pltpu.async_copy(table_hbm.at[idx_v.at[j + NBUF]], bufs[b],
                                 gsems[b])

        return carry

    lax.fori_loop(0, NG // NBUF, outer, 0)


def kernel(input, embedding):
    B, H = input.shape
    idx = input.astype(jnp.int32).reshape(NW, NG, G)
    out = _gather_kernel(idx, embedding)
    return out.reshape(B, H, N_EMBED)


# speedup vs baseline: 5.9462x; 1.7799x over previous
import functools

import jax
import jax.numpy as jnp
from jax import lax
from jax.experimental import pallas as pl
from jax.experimental.pallas import tpu as pltpu
from jax.experimental.pallas import tpu_sc as plsc

N_EMBED = 128
BATCH = 4096
HIST = 50
NC = 2
NS = 16
NW = NC * NS
BPW = BATCH // NW
K = 8
NSB = BPW // K
NBUF = 2

_mesh = plsc.VectorSubcoreMesh(core_axis_name="c", subcore_axis_name="s")


@functools.partial(
    pl.kernel,
    out_type=jax.ShapeDtypeStruct((BATCH, HIST, N_EMBED), jnp.float32),
    mesh=_mesh,
    scratch_types=[
        pltpu.VMEM((BPW, HIST), jnp.int32),
        [pltpu.VMEM((K, HIST, N_EMBED), jnp.float32) for _ in range(NBUF)],
        [pltpu.SemaphoreType.DMA for _ in range(NBUF)],
        [pltpu.SemaphoreType.DMA for _ in range(NBUF)],
    ],
)
def _gather_kernel(idx_hbm, table_hbm, out_hbm, idx_v, bufs, gsems, psems):
    wid = lax.axis_index("s") * NC + lax.axis_index("c")
    b0 = wid * BPW
    pltpu.sync_copy(idx_hbm.at[pl.ds(b0, BPW)], idx_v)

    def fire(s, r):
        for i in range(K):
            pltpu.async_copy(table_hbm.at[idx_v.at[s * K + i]], bufs[r].at[i],
                             gsems[r])

    def drain(s, r):
        for i in range(K):
            pltpu.make_async_copy(table_hbm.at[idx_v.at[s * K + i]],
                                  bufs[r].at[i], gsems[r]).wait()

    for r in range(NBUF):
        fire(r, r)

    def outer(t, carry):
        for r in range(NBUF):
            s = t * NBUF + r
            drain(s, r)
            pltpu.async_copy(bufs[r], out_hbm.at[pl.ds(b0 + s * K, K)],
                             psems[r]).wait()

            @pl.when(s + NBUF < NSB)
            def _():
                fire(s + NBUF, r)

        return carry

    lax.fori_loop(0, NSB // NBUF, outer, 0)


def kernel(input, embedding):
    return _gather_kernel(input.astype(jnp.int32), embedding)
